# Initial kernel scaffold; baseline (speedup 1.0000x reference)
import numpy as np
import jax
import jax.numpy as jnp
from jax.experimental import pallas as pl

_T = 6
_NP = 16
_NN = 32
_NE = 16
_P96 = _T * _NP


def _build_static():
    C = np.zeros((_NP, _T, _P96), np.float32)
    qmap = np.zeros((_NP, _T), np.int64)
    for i in range(_T):
        Ci = i + 1

        def tf(cp, npp):
            m = cp * 16 + npp
            return (m % Ci) * 16 + (m // Ci)

        for j in range(_NP):
            C[j, i, tf(Ci - 1, j)] += 1.0
        il = 2
        for t in range(i, -1, -1):
            for j in range(_NP):
                for k in range(-il + 1, il):
                    for l in range(-il + 1, il):
                        idx = j + 16 * k + l
                        if (not (j == 0 and l == 0 and il == 2)) and 0 <= idx < _NP:
                            C[j, i, tf(t, idx)] += 1.0
            il += 1
        for j in range(_NP):
            m = i * 16 + j
            qmap[j, i] = (m % _T) * 16 + (m // _T)
    return C, qmap


_C_COUNTS, _QMAP = _build_static()


def _body(pq_ref, wqt_ref, p_ref, pt_ref, wk_ref, bk_ref, wvt_ref, bv_ref,
          c_ref, out_ref):
    qj = jnp.dot(pq_ref[0], wqt_ref[...], preferred_element_type=jnp.float32)
    kjt = jnp.dot(wk_ref[0], pt_ref[...], preferred_element_type=jnp.float32)
    kjt = kjt + bk_ref[0].reshape(_NN, 1)
    s = jnp.dot(qj, kjt, preferred_element_type=jnp.float32)
    w = c_ref[0] * s
    vj = jnp.dot(p_ref[...], wvt_ref[0], preferred_element_type=jnp.float32)
    vj = vj + bv_ref[0].reshape(1, _NN)
    out_ref[0] = jnp.dot(w, vj, preferred_element_type=jnp.float32)


def kernel(x, Wq, Wk, bk, Wv, bv):
    P = x[0].reshape(_T, 4, 4, 4, 4).transpose(0, 1, 3, 2, 4).reshape(_P96, _NE)
    Pq = P[_QMAP.reshape(-1)].reshape(_NP, _T, _NE)
    PT = P.T
    WqT = Wq.T
    WvT = Wv.transpose(0, 2, 1)
    C = jnp.asarray(_C_COUNTS)

    out = pl.pallas_call(
        _body,
        grid=(_NP,),
        in_specs=[
            pl.BlockSpec((1, _T, _NE), lambda j: (j, 0, 0)),
            pl.BlockSpec((_NE, _NN), lambda j: (0, 0)),
            pl.BlockSpec((_P96, _NE), lambda j: (0, 0)),
            pl.BlockSpec((_NE, _P96), lambda j: (0, 0)),
            pl.BlockSpec((1, _NN, _NE), lambda j: (j, 0, 0)),
            pl.BlockSpec((1, _NN), lambda j: (j, 0)),
            pl.BlockSpec((1, _NE, _NN), lambda j: (j, 0, 0)),
            pl.BlockSpec((1, _NN), lambda j: (j, 0)),
            pl.BlockSpec((1, _T, _P96), lambda j: (j, 0, 0)),
        ],
        out_specs=pl.BlockSpec((1, _T, _NN), lambda j: (j, 0, 0)),
        out_shape=jax.ShapeDtypeStruct((_NP, _T, _NN), jnp.float32),
    )(Pq, WqT, P, PT, Wk, bk, WvT, bv, C)

    return out.transpose(1, 0, 2)[None]


# speedup vs baseline: 7.8530x; 7.8530x over previous
import numpy as np
import jax
import jax.numpy as jnp
from jax.experimental import pallas as pl

_T = 6
_NP = 16
_NN = 32
_NE = 16
_P96 = _T * _NP


def _build_static():
    C = np.zeros((_NP, _T, _P96), np.float32)
    qmap = np.zeros((_NP, _T), np.int64)
    for i in range(_T):
        Ci = i + 1

        def tf(cp, npp):
            m = cp * 16 + npp
            return (m % Ci) * 16 + (m // Ci)

        for j in range(_NP):
            C[j, i, tf(Ci - 1, j)] += 1.0
        il = 2
        for t in range(i, -1, -1):
            for j in range(_NP):
                for k in range(-il + 1, il):
                    for l in range(-il + 1, il):
                        idx = j + 16 * k + l
                        if (not (j == 0 and l == 0 and il == 2)) and 0 <= idx < _NP:
                            C[j, i, tf(t, idx)] += 1.0
            il += 1
        for j in range(_NP):
            m = i * 16 + j
            qmap[j, i] = (m % _T) * 16 + (m // _T)
    return C, qmap


_C_COUNTS, _QMAP = _build_static()


def _body(pq_ref, wqt_ref, p_ref, pt_ref, wk_ref, bk_ref, wvt_ref, bv_ref,
          c_ref, out_ref):
    qj = jnp.dot(pq_ref[0], wqt_ref[...], preferred_element_type=jnp.float32)
    kjt = jnp.dot(wk_ref[0], pt_ref[...], preferred_element_type=jnp.float32)
    kjt = kjt + bk_ref[0, 0].reshape(_NN, 1)
    s = jnp.dot(qj, kjt, preferred_element_type=jnp.float32)
    w = c_ref[0] * s
    vj = jnp.dot(p_ref[...], wvt_ref[0], preferred_element_type=jnp.float32)
    vj = vj + bv_ref[0, 0].reshape(1, _NN)
    out_ref[0] = jnp.dot(w, vj, preferred_element_type=jnp.float32)


def kernel(x, Wq, Wk, bk, Wv, bv):
    P = x[0].reshape(_T, 4, 4, 4, 4).transpose(0, 1, 3, 2, 4).reshape(_P96, _NE)
    Pq = P[_QMAP.reshape(-1)].reshape(_NP, _T, _NE)
    PT = P.T
    WqT = Wq.T
    WvT = Wv.transpose(0, 2, 1)
    C = jnp.asarray(_C_COUNTS)

    out = pl.pallas_call(
        _body,
        grid=(_NP,),
        in_specs=[
            pl.BlockSpec((1, _T, _NE), lambda j: (j, 0, 0)),
            pl.BlockSpec((_NE, _NN), lambda j: (0, 0)),
            pl.BlockSpec((_P96, _NE), lambda j: (0, 0)),
            pl.BlockSpec((_NE, _P96), lambda j: (0, 0)),
            pl.BlockSpec((1, _NN, _NE), lambda j: (j, 0, 0)),
            pl.BlockSpec((1, 1, _NN), lambda j: (j, 0, 0)),
            pl.BlockSpec((1, _NE, _NN), lambda j: (j, 0, 0)),
            pl.BlockSpec((1, 1, _NN), lambda j: (j, 0, 0)),
            pl.BlockSpec((1, _T, _P96), lambda j: (j, 0, 0)),
        ],
        out_specs=pl.BlockSpec((1, _T, _NN), lambda j: (j, 0, 0)),
        out_shape=jax.ShapeDtypeStruct((_NP, _T, _NN), jnp.float32),
    )(Pq, WqT, P, PT, Wk, bk[:, None, :], WvT, bv[:, None, :], C)

    return out.transpose(1, 0, 2)[None]
